# Initial kernel scaffold; baseline (speedup 1.0000x reference)
import jax
import jax.numpy as jnp
from jax.experimental import pallas as pl


def kernel(features, Wq, Wk, Wv, Wo, ln1_g, ln1_b, W1, b1, W2, b2, ln2_g, ln2_b):
    raise NotImplementedError("write your pallas kernel here")


# speedup vs baseline: 2.5037x; 2.5037x over previous
import functools

import numpy as np
import jax
import jax.numpy as jnp
from jax.experimental import pallas as pl

DEPTH = 2
HEADS = 8
BUCKET = 64
SUB = 2
FEAT = 1024
SEQ = 960
BATCH = 2
DIM = FEAT // SUB
DFF = 4 * DIM
S2 = SEQ * SUB
DH = DIM // HEADS

TOK = 640
NCHUNK = S2 // TOK
NB = TOK // BUCKET


def _make_pe():
    pe = np.zeros((1000, FEAT), dtype=np.float32)
    position = np.arange(1000, dtype=np.float32)[:, None]
    dimension = np.arange(FEAT, dtype=np.float32)
    div_term = 10000.0 ** (2.0 * dimension / FEAT)
    pe[:, 0::2] = np.sin(position / div_term[0::2])
    pe[:, 1::2] = np.cos(position / div_term[1::2])
    return pe[:SEQ].reshape(S2, DIM)


_PE = _make_pe()


def _layer_norm(x, g, b):
    mean = jnp.mean(x, axis=-1, keepdims=True)
    xc = x - mean
    std = jnp.sqrt(jnp.sum(xc * xc, axis=-1, keepdims=True) / (DIM - 1))
    return g * xc / (std + 1e-6) + b


def _fused_kernel(x_ref, pe_ref, wq_ref, wk_ref, wv_ref, wo_ref,
                  g1_ref, bb1_ref, w1_ref, b1_ref, w2_ref, b2_ref,
                  g2_ref, bb2_ref, out_ref):
    c = pl.program_id(1)
    x = x_ref[0] + pe_ref[...]

    rows = jax.lax.broadcasted_iota(jnp.int32, (BUCKET, BUCKET), 0)
    cols = jax.lax.broadcasted_iota(jnp.int32, (BUCKET, BUCKET), 1)
    causal = (rows >= cols)[None]

    for l in range(DEPTH):
        h = _layer_norm(x, g1_ref[l], bb1_ref[l])
        q = jnp.dot(h, wq_ref[l], preferred_element_type=jnp.float32)
        k = jnp.dot(h, wk_ref[l], preferred_element_type=jnp.float32)
        v = jnp.dot(h, wv_ref[l], preferred_element_type=jnp.float32)

        def heads(a):
            return a.reshape(NB, BUCKET, HEADS, DH).transpose(0, 2, 1, 3).reshape(
                NB * HEADS, BUCKET, DH)

        qb, kb, vb = heads(q), heads(k), heads(v)
        s = jax.lax.dot_general(
            qb, kb, (((2,), (2,)), ((0,), (0,))),
            preferred_element_type=jnp.float32) * (1.0 / 8.0)
        s = jnp.where(causal, s, -1e9)
        w = jax.nn.softmax(s, axis=-1)
        o = jax.lax.dot_general(
            w, vb, (((2,), (1,)), ((0,), (0,))),
            preferred_element_type=jnp.float32)
        o = o.reshape(NB, HEADS, BUCKET, DH).transpose(0, 2, 1, 3).reshape(TOK, DIM)
        x = x + jnp.dot(o, wo_ref[l], preferred_element_type=jnp.float32)

        h2 = _layer_norm(x, g2_ref[l], bb2_ref[l])
        ff = jax.nn.gelu(
            jnp.dot(h2, w1_ref[l], preferred_element_type=jnp.float32)
            + b1_ref[l])
        ff = jnp.dot(ff, w2_ref[l], preferred_element_type=jnp.float32) + b2_ref[l]
        x = x + ff

    part = jnp.sum(x.reshape(TOK // 2, 2, DIM), axis=0) * (1.0 / SEQ)
    part = part.reshape(1, 1, FEAT)

    @pl.when(c == 0)
    def _():
        out_ref[...] = part

    @pl.when(c > 0)
    def _():
        out_ref[...] += part


@functools.partial(jax.jit, static_argnames=())
def kernel(features, Wq, Wk, Wv, Wo, ln1_g, ln1_b, W1, b1, W2, b2, ln2_g, ln2_b):
    x = features.reshape(BATCH, S2, DIM)
    pe = jnp.asarray(_PE)

    full = lambda shape: pl.BlockSpec(shape, lambda b, c: (0,) * len(shape))
    grid_spec = pl.GridSpec(
        grid=(BATCH, NCHUNK),
        in_specs=[
            pl.BlockSpec((1, TOK, DIM), lambda b, c: (b, c, 0)),
            pl.BlockSpec((TOK, DIM), lambda b, c: (c, 0)),
            full((DEPTH, DIM, DIM)),
            full((DEPTH, DIM, DIM)),
            full((DEPTH, DIM, DIM)),
            full((DEPTH, DIM, DIM)),
            full((DEPTH, DIM)),
            full((DEPTH, DIM)),
            full((DEPTH, DIM, DFF)),
            full((DEPTH, DFF)),
            full((DEPTH, DFF, DIM)),
            full((DEPTH, DIM)),
            full((DEPTH, DIM)),
            full((DEPTH, DIM)),
        ],
        out_specs=pl.BlockSpec((1, 1, FEAT), lambda b, c: (b, 0, 0)),
    )
    pooled = pl.pallas_call(
        _fused_kernel,
        grid_spec=grid_spec,
        out_shape=jax.ShapeDtypeStruct((BATCH, 1, FEAT), jnp.float32),
    )(x, pe, Wq, Wk, Wv, Wo, ln1_g, ln1_b, W1, b1, W2, b2, ln2_g, ln2_b)
    return pooled.reshape(BATCH, FEAT)
